# baseline (speedup 1.0000x reference)
import dataclasses
import functools

import jax
import jax.numpy as jnp
from jax import lax
from jax.experimental import pallas as pl
from jax.experimental.pallas import tpu as pltpu
from jax.experimental.pallas import tpu_sc as plsc

N_NODES = 10000
D = 128
N_EDGES = 320000

NC = 2
NS = 16
NW = NC * NS

EDGE_BATCH = 64
E_PAD = 327680
ROWS_PER_W = E_PAD // (NW * EDGE_BATCH)
N_PAD = 10240
ROWS_PER_SUB = N_PAD // NS
PAD_DST = 10008
G_ROWS = 32
R_CORE0 = 288
R_CORE1 = 32

_vector_mesh = plsc.VectorSubcoreMesh(core_axis_name="c", subcore_axis_name="s")

_sc_params = pltpu.CompilerParams()
if "needs_layout_passes" in pltpu.CompilerParams.__dataclass_fields__:
    _sc_params = dataclasses.replace(_sc_params, needs_layout_passes=False)


def _hist_body(dst_hbm, histp_hbm, dst_v, hist_v):
    c = lax.axis_index("c")
    s = lax.axis_index("s")
    wid = s * NC + c

    zeros16 = jnp.zeros((16,), jnp.float32)
    ones16 = jnp.ones((16,), jnp.float32)

    @pl.loop(0, N_PAD // 16)
    def _(i):
        hist_v[pl.ds(i * 16, 16)] = zeros16

    pltpu.sync_copy(dst_hbm.at[pl.ds(wid * ROWS_PER_W, ROWS_PER_W)], dst_v)

    @pl.loop(0, ROWS_PER_W)
    def _(j):
        @pl.loop(0, EDGE_BATCH // 16)
        def _(k):
            idx = dst_v[j, pl.ds(k * 16, 16)]
            plsc.addupdate_scatter(hist_v, [idx], ones16)

    pltpu.sync_copy(hist_v, histp_hbm.at[wid])


@jax.jit
def _sc_hist(dst2d):
    kern = pl.kernel(
        _hist_body,
        out_type=jax.ShapeDtypeStruct((NW, N_PAD), jnp.float32),
        mesh=_vector_mesh,
        scratch_types=[
            pltpu.VMEM((ROWS_PER_W, EDGE_BATCH), jnp.int32),
            pltpu.VMEM((N_PAD,), jnp.float32),
        ],
        compiler_params=_sc_params,
    )
    return kern(dst2d)


BLK = 1024
GRID = 10


def _in_deg_bcast(hist):
    ones_w = jnp.ones((NW, D), jnp.float32)
    return lax.dot_general(hist, ones_w, (((0,), (0,)), ((), ())),
                           preferred_element_type=jnp.float32,
                           precision=lax.Precision.HIGHEST)


def _pre_body(hist_ref, x_ref, xs_ref):
    in_deg = _in_deg_bcast(hist_ref[...])
    rs = lax.rsqrt(jnp.maximum(in_deg, 1.0))
    xs_ref[...] = rs * x_ref[...]


@jax.jit
def _tc_pre(histp, x):
    return pl.pallas_call(
        _pre_body,
        grid=(GRID,),
        in_specs=[
            pl.BlockSpec((NW, BLK), lambda i: (0, i)),
            pl.BlockSpec((BLK, D), lambda i: (i, 0)),
        ],
        out_specs=pl.BlockSpec((BLK, D), lambda i: (i, 0)),
        out_shape=jax.ShapeDtypeStruct((N_NODES, D), jnp.float32),
    )(histp, x)


def _agg_body(xs_hbm, src_hbm, dst_hbm, zeros_hbm, accp_hbm,
              src_v, dst_v, rows_v, acc_sh,
              gsem0, gsem1, gsem2, gsem3, ssem0, ssem1, ssem2, ssem3):
    c = lax.axis_index("c")
    s = lax.axis_index("s")
    my_rows = jnp.where(c == 0, R_CORE0, R_CORE1)
    row0 = c * (NS * R_CORE0) + s * my_rows
    my_groups = my_rows // G_ROWS

    pltpu.sync_copy(zeros_hbm, acc_sh.at[pl.ds(s * ROWS_PER_SUB, ROWS_PER_SUB)])
    plsc.subcore_barrier()

    gsem = (gsem0, gsem1, gsem2, gsem3)
    ssem = (ssem0, ssem1, ssem2, ssem3)

    def start_gather(b, j):
        pltpu.async_copy(xs_hbm.at[src_v.at[j]], rows_v.at[b], gsem[b])

    def wait_gather(b):
        pltpu.make_async_copy(xs_hbm.at[src_v.at[0]], rows_v.at[b],
                              gsem[b]).wait()

    def start_scatter(b, j):
        pltpu.async_copy(rows_v.at[b], acc_sh.at[dst_v.at[j]], ssem[b],
                         add=True)

    def wait_scatter(b):
        pltpu.make_async_copy(rows_v.at[b], acc_sh.at[dst_v.at[0]],
                              ssem[b]).wait()

    @pl.loop(0, my_groups)
    def _(g):
        base = row0 + g * G_ROWS
        pltpu.sync_copy(src_hbm.at[pl.ds(base, G_ROWS)], src_v)
        pltpu.sync_copy(dst_hbm.at[pl.ds(base, G_ROWS)], dst_v)
        start_gather(0, 0)
        start_gather(1, 1)

        @pl.loop(0, G_ROWS // 4)
        def _(it):
            j0 = it * 4
            for b in range(4):
                j = j0 + b
                wait_gather(b)
                start_scatter(b, j)
                bpf = (b + 2) % 4
                pf = j + 2

                @pl.when(pf < G_ROWS)
                def _():
                    @pl.when(pf >= 4)
                    def _():
                        wait_scatter(bpf)

                    start_gather(bpf, pf)

        for b in range(4):
            wait_scatter(b)

    plsc.subcore_barrier()
    pltpu.sync_copy(acc_sh.at[pl.ds(s * ROWS_PER_SUB, ROWS_PER_SUB)],
                    accp_hbm.at[c, pl.ds(s * ROWS_PER_SUB, ROWS_PER_SUB)])


@jax.jit
def _sc_agg(xs, src2d, dst2d, zeros_blk):
    kern = pl.kernel(
        _agg_body,
        out_type=jax.ShapeDtypeStruct((NC, N_PAD, D), jnp.float32),
        mesh=_vector_mesh,
        scratch_types=[
            pltpu.VMEM((G_ROWS, EDGE_BATCH), jnp.int32),
            pltpu.VMEM((G_ROWS, EDGE_BATCH), jnp.int32),
            pltpu.VMEM((4, EDGE_BATCH, D), jnp.float32),
            pltpu.VMEM_SHARED((N_PAD, D), jnp.float32),
            pltpu.SemaphoreType.DMA,
            pltpu.SemaphoreType.DMA,
            pltpu.SemaphoreType.DMA,
            pltpu.SemaphoreType.DMA,
            pltpu.SemaphoreType.DMA,
            pltpu.SemaphoreType.DMA,
            pltpu.SemaphoreType.DMA,
            pltpu.SemaphoreType.DMA,
        ],
    )
    return kern(xs, src2d, dst2d, zeros_blk)


def _post_body(accp_ref, hist_ref, x_ref, wt_ref, b_ref, out_ref):
    in_deg = _in_deg_bcast(hist_ref[...])
    rs = lax.rsqrt(jnp.maximum(in_deg, 1.0))
    acc = accp_ref[0] + accp_ref[1]
    x = x_ref[...]
    msum = rs * acc
    inv = 1.0 / (in_deg + 1.0)
    h = jnp.where(in_deg > 0.0, (msum + x) * inv, x)
    y = jnp.dot(h, wt_ref[...], preferred_element_type=jnp.float32,
                precision=lax.Precision.HIGHEST)
    out_ref[...] = jnp.maximum(y + b_ref[...], 0.0)


@jax.jit
def _tc_post(accp, histp, x, wt, b2d):
    return pl.pallas_call(
        _post_body,
        grid=(GRID,),
        in_specs=[
            pl.BlockSpec((NC, BLK, D), lambda i: (0, i, 0)),
            pl.BlockSpec((NW, BLK), lambda i: (0, i)),
            pl.BlockSpec((BLK, D), lambda i: (i, 0)),
            pl.BlockSpec((D, D), lambda i: (0, 0)),
            pl.BlockSpec((1, D), lambda i: (0, 0)),
        ],
        out_specs=pl.BlockSpec((BLK, D), lambda i: (i, 0)),
        out_shape=jax.ShapeDtypeStruct((N_NODES, D), jnp.float32),
    )(accp, histp, x, wt, b2d)


@jax.jit
def kernel(features, edge_index, W, b):
    src = edge_index[0].astype(jnp.int32)
    dst = edge_index[1].astype(jnp.int32)
    n_pad_edges = E_PAD - N_EDGES
    src2d = jnp.concatenate(
        [src, jnp.zeros((n_pad_edges,), jnp.int32)]).reshape(-1, EDGE_BATCH)
    dst2d = jnp.concatenate(
        [dst, jnp.full((n_pad_edges,), PAD_DST, jnp.int32)]).reshape(-1, EDGE_BATCH)

    histp = _sc_hist(dst2d)
    xs = _tc_pre(histp, features)
    zeros_blk = jnp.zeros((ROWS_PER_SUB, D), jnp.float32)
    accp = _sc_agg(xs, src2d, dst2d, zeros_blk)
    return _tc_post(accp, histp, features, W.T, b[None, :])


# speedup vs baseline: 1.0102x; 1.0102x over previous
import dataclasses
import functools

import jax
import jax.numpy as jnp
from jax import lax
from jax.experimental import pallas as pl
from jax.experimental.pallas import tpu as pltpu
from jax.experimental.pallas import tpu_sc as plsc

N_NODES = 10000
D = 128
N_EDGES = 320000

NC = 2
NS = 16
NW = NC * NS

EDGE_BATCH = 64
E_PAD = 327680
ROWS_PER_W = E_PAD // (NW * EDGE_BATCH)
N_PAD = 10240
ROWS_PER_SUB = N_PAD // NS
PAD_DST = 10008
G_ROWS = 8
R_CORE0 = 296
R_CORE1 = 24

_vector_mesh = plsc.VectorSubcoreMesh(core_axis_name="c", subcore_axis_name="s")

_sc_params = pltpu.CompilerParams()
if "needs_layout_passes" in pltpu.CompilerParams.__dataclass_fields__:
    _sc_params = dataclasses.replace(_sc_params, needs_layout_passes=False)


def _hist_body(dst_hbm, histp_hbm, dst_v, hist_v):
    c = lax.axis_index("c")
    s = lax.axis_index("s")
    wid = s * NC + c

    zeros16 = jnp.zeros((16,), jnp.float32)
    ones16 = jnp.ones((16,), jnp.float32)

    @pl.loop(0, N_PAD // 16)
    def _(i):
        hist_v[pl.ds(i * 16, 16)] = zeros16

    pltpu.sync_copy(dst_hbm.at[pl.ds(wid * ROWS_PER_W, ROWS_PER_W)], dst_v)

    @pl.loop(0, ROWS_PER_W)
    def _(j):
        @pl.loop(0, EDGE_BATCH // 16)
        def _(k):
            idx = dst_v[j, pl.ds(k * 16, 16)]
            plsc.addupdate_scatter(hist_v, [idx], ones16)

    pltpu.sync_copy(hist_v, histp_hbm.at[wid])


@jax.jit
def _sc_hist(dst2d):
    kern = pl.kernel(
        _hist_body,
        out_type=jax.ShapeDtypeStruct((NW, N_PAD), jnp.float32),
        mesh=_vector_mesh,
        scratch_types=[
            pltpu.VMEM((ROWS_PER_W, EDGE_BATCH), jnp.int32),
            pltpu.VMEM((N_PAD,), jnp.float32),
        ],
        compiler_params=_sc_params,
    )
    return kern(dst2d)


BLK = 1024
GRID = 10


def _in_deg_bcast(hist):
    ones_w = jnp.ones((NW, D), jnp.float32)
    return lax.dot_general(hist, ones_w, (((0,), (0,)), ((), ())),
                           preferred_element_type=jnp.float32,
                           precision=lax.Precision.HIGHEST)


def _pre_body(hist_ref, x_ref, xs_ref):
    in_deg = _in_deg_bcast(hist_ref[...])
    rs = lax.rsqrt(jnp.maximum(in_deg, 1.0))
    xs_ref[...] = rs * x_ref[...]


@jax.jit
def _tc_pre(histp, x):
    return pl.pallas_call(
        _pre_body,
        grid=(GRID,),
        in_specs=[
            pl.BlockSpec((NW, BLK), lambda i: (0, i)),
            pl.BlockSpec((BLK, D), lambda i: (i, 0)),
        ],
        out_specs=pl.BlockSpec((BLK, D), lambda i: (i, 0)),
        out_shape=jax.ShapeDtypeStruct((N_NODES, D), jnp.float32),
    )(histp, x)


def _agg_body(xs_hbm, src_hbm, dst_hbm, zeros_hbm, accp_hbm,
              src_v, dst_v, rows_v, acc_sh,
              gsem0, gsem1, gsem2, gsem3, ssem0, ssem1, ssem2, ssem3):
    c = lax.axis_index("c")
    s = lax.axis_index("s")
    my_rows = jnp.where(c == 0, R_CORE0, R_CORE1)
    row0 = c * (NS * R_CORE0) + s * my_rows
    my_groups = my_rows // G_ROWS

    pltpu.sync_copy(zeros_hbm, acc_sh.at[pl.ds(s * ROWS_PER_SUB, ROWS_PER_SUB)])
    plsc.subcore_barrier()

    gsem = (gsem0, gsem1, gsem2, gsem3)
    ssem = (ssem0, ssem1, ssem2, ssem3)

    def start_gather(b, j):
        pltpu.async_copy(xs_hbm.at[src_v.at[j]], rows_v.at[b], gsem[b])

    def wait_gather(b):
        pltpu.make_async_copy(xs_hbm.at[src_v.at[0]], rows_v.at[b],
                              gsem[b]).wait()

    def start_scatter(b, j):
        pltpu.async_copy(rows_v.at[b], acc_sh.at[dst_v.at[j]], ssem[b],
                         add=True)

    def wait_scatter(b):
        pltpu.make_async_copy(rows_v.at[b], acc_sh.at[dst_v.at[0]],
                              ssem[b]).wait()

    @pl.loop(0, my_groups)
    def _(g):
        base = row0 + g * G_ROWS
        pltpu.sync_copy(src_hbm.at[pl.ds(base, G_ROWS)], src_v)
        pltpu.sync_copy(dst_hbm.at[pl.ds(base, G_ROWS)], dst_v)
        start_gather(0, 0)
        start_gather(1, 1)

        @pl.loop(0, G_ROWS // 4)
        def _(it):
            j0 = it * 4
            for b in range(4):
                j = j0 + b
                wait_gather(b)
                start_scatter(b, j)
                bpf = (b + 2) % 4
                pf = j + 2

                @pl.when(pf < G_ROWS)
                def _():
                    @pl.when(pf >= 4)
                    def _():
                        wait_scatter(bpf)

                    start_gather(bpf, pf)

        for b in range(4):
            wait_scatter(b)

    plsc.subcore_barrier()
    pltpu.sync_copy(acc_sh.at[pl.ds(s * ROWS_PER_SUB, ROWS_PER_SUB)],
                    accp_hbm.at[c, pl.ds(s * ROWS_PER_SUB, ROWS_PER_SUB)])


@jax.jit
def _sc_agg(xs, src2d, dst2d, zeros_blk):
    kern = pl.kernel(
        _agg_body,
        out_type=jax.ShapeDtypeStruct((NC, N_PAD, D), jnp.float32),
        mesh=_vector_mesh,
        scratch_types=[
            pltpu.VMEM((G_ROWS, EDGE_BATCH), jnp.int32),
            pltpu.VMEM((G_ROWS, EDGE_BATCH), jnp.int32),
            pltpu.VMEM((4, EDGE_BATCH, D), jnp.float32),
            pltpu.VMEM_SHARED((N_PAD, D), jnp.float32),
            pltpu.SemaphoreType.DMA,
            pltpu.SemaphoreType.DMA,
            pltpu.SemaphoreType.DMA,
            pltpu.SemaphoreType.DMA,
            pltpu.SemaphoreType.DMA,
            pltpu.SemaphoreType.DMA,
            pltpu.SemaphoreType.DMA,
            pltpu.SemaphoreType.DMA,
        ],
    )
    return kern(xs, src2d, dst2d, zeros_blk)


def _post_body(accp_ref, hist_ref, x_ref, wt_ref, b_ref, out_ref):
    in_deg = _in_deg_bcast(hist_ref[...])
    rs = lax.rsqrt(jnp.maximum(in_deg, 1.0))
    acc = accp_ref[0] + accp_ref[1]
    x = x_ref[...]
    msum = rs * acc
    inv = 1.0 / (in_deg + 1.0)
    h = jnp.where(in_deg > 0.0, (msum + x) * inv, x)
    y = jnp.dot(h, wt_ref[...], preferred_element_type=jnp.float32,
                precision=lax.Precision.HIGHEST)
    out_ref[...] = jnp.maximum(y + b_ref[...], 0.0)


@jax.jit
def _tc_post(accp, histp, x, wt, b2d):
    return pl.pallas_call(
        _post_body,
        grid=(GRID,),
        in_specs=[
            pl.BlockSpec((NC, BLK, D), lambda i: (0, i, 0)),
            pl.BlockSpec((NW, BLK), lambda i: (0, i)),
            pl.BlockSpec((BLK, D), lambda i: (i, 0)),
            pl.BlockSpec((D, D), lambda i: (0, 0)),
            pl.BlockSpec((1, D), lambda i: (0, 0)),
        ],
        out_specs=pl.BlockSpec((BLK, D), lambda i: (i, 0)),
        out_shape=jax.ShapeDtypeStruct((N_NODES, D), jnp.float32),
    )(accp, histp, x, wt, b2d)


@jax.jit
def kernel(features, edge_index, W, b):
    src = edge_index[0].astype(jnp.int32)
    dst = edge_index[1].astype(jnp.int32)
    n_pad_edges = E_PAD - N_EDGES
    src2d = jnp.concatenate(
        [src, jnp.zeros((n_pad_edges,), jnp.int32)]).reshape(-1, EDGE_BATCH)
    dst2d = jnp.concatenate(
        [dst, jnp.full((n_pad_edges,), PAD_DST, jnp.int32)]).reshape(-1, EDGE_BATCH)

    histp = _sc_hist(dst2d)
    xs = _tc_pre(histp, features)
    zeros_blk = jnp.zeros((ROWS_PER_SUB, D), jnp.float32)
    accp = _sc_agg(xs, src2d, dst2d, zeros_blk)
    return _tc_post(accp, histp, features, W.T, b[None, :])
